# baseline (speedup 1.0000x reference)
import functools

import jax
import jax.numpy as jnp
from jax import lax
from jax.experimental import pallas as pl
from jax.experimental.pallas import tpu as pltpu
from jax.experimental.pallas import tpu_sc as plsc

N = 10000
E = 320000
D = 128
H = 64
O = 3
G = 64

NC = 2
NS = 16
NW = NC * NS

CH = 256
NCH = 40
NCH1 = NCH + 1
EPW = NCH * CH
EPAD = NW * EPW

NPAD = 10240
RPT = NPAD // NS
RB = 1024
NRB = NPAD // RB
DEGW = 16

_mesh = plsc.VectorSubcoreMesh(core_axis_name="c", subcore_axis_name="s",
                               num_cores=NC, num_subcores=NS)
_sc_params = pltpu.CompilerParams(use_tc_tiling_on_sc=False)


@functools.partial(
    pl.kernel,
    out_type=jax.ShapeDtypeStruct((NC, NPAD, DEGW), jnp.float32),
    mesh=_mesh,
    compiler_params=_sc_params,
    scratch_types=[
        pltpu.VMEM((NCH1, CH), jnp.int32),
        pltpu.VMEM((CH, DEGW), jnp.float32),
        pltpu.VMEM_SHARED((NPAD, DEGW), jnp.float32),
    ],
)
def _sc_degree(col_hbm, ones_hbm, zeros_hbm, out_hbm, col_v, ones_v, acc_sh):
    c = lax.axis_index("c")
    s = lax.axis_index("s")
    wid = s * NC + c
    pltpu.sync_copy(zeros_hbm, acc_sh.at[pl.ds(s * RPT, RPT)])
    pltpu.sync_copy(ones_hbm, ones_v)
    pltpu.sync_copy(col_hbm.at[wid], col_v)
    plsc.subcore_barrier()

    def body(j, carry):
        pltpu.sync_copy(ones_v, acc_sh.at[col_v.at[j]], add=True)
        return carry

    lax.fori_loop(0, NCH, body, 0)
    plsc.subcore_barrier()
    pltpu.sync_copy(acc_sh.at[pl.ds(s * RPT, RPT)],
                    out_hbm.at[c, pl.ds(s * RPT, RPT)])


@functools.partial(
    pl.kernel,
    out_type=jax.ShapeDtypeStruct((NC, NPAD, H), jnp.float32),
    mesh=_mesh,
    compiler_params=_sc_params,
    scratch_types=[
        pltpu.VMEM((NCH1, CH), jnp.int32),
        pltpu.VMEM((NCH1, CH), jnp.int32),
        pltpu.VMEM((2, CH, H), jnp.float32),
        pltpu.VMEM_SHARED((NPAD, H), jnp.float32),
        pltpu.SemaphoreType.DMA,
    ],
)
def _sc_aggregate(y_hbm, row_hbm, col_hbm, zeros_hbm, out_hbm,
                  row_v, col_v, buf2, acc_sh, sem):
    c = lax.axis_index("c")
    s = lax.axis_index("s")
    wid = s * NC + c
    pltpu.sync_copy(zeros_hbm, acc_sh.at[pl.ds(s * RPT, RPT)])
    pltpu.sync_copy(row_hbm.at[wid], row_v)
    pltpu.sync_copy(col_hbm.at[wid], col_v)
    plsc.subcore_barrier()

    pltpu.async_copy(y_hbm.at[row_v.at[0]], buf2.at[0], sem).wait()

    def body(j, carry):
        d = pltpu.async_copy(y_hbm.at[row_v.at[j + 1]],
                             buf2.at[lax.rem(j + 1, 2)], sem)
        pltpu.sync_copy(buf2.at[lax.rem(j, 2)], acc_sh.at[col_v.at[j]],
                        add=True)
        d.wait()
        return carry

    lax.fori_loop(0, NCH, body, 0)
    plsc.subcore_barrier()
    pltpu.sync_copy(acc_sh.at[pl.ds(s * RPT, RPT)],
                    out_hbm.at[c, pl.ds(s * RPT, RPT)])


def _dinv_block(p0, p1):
    deg = 1.0 + p0[:, 0:1] + p1[:, 0:1]
    return lax.rsqrt(deg)


def _tc_scale_matmul_body(x_ref, p0_ref, p1_ref, w_ref, o_ref):
    dinv = _dinv_block(p0_ref, p1_ref)
    xl = jnp.dot(x_ref[...], w_ref[...], preferred_element_type=jnp.float32)
    o_ref[...] = xl * dinv


def _tc_fuse_body(q0_ref, q1_ref, y_ref, p0_ref, p1_ref, b_ref, w_ref, o_ref):
    dinv = _dinv_block(p0_ref, p1_ref)
    h = dinv * (q0_ref[...] + q1_ref[...] + y_ref[...]) + b_ref[...]
    h = jnp.maximum(h, 0.0)
    o_ref[...] = jnp.dot(h, w_ref[...], preferred_element_type=jnp.float32) * dinv


def _tc_final_body(q0_ref, q1_ref, y_ref, p0_ref, p1_ref, bt_ref, b_ref,
                   fcw_ref, fcb_ref, o_ref, acc):
    i = pl.program_id(0)

    @pl.when(i == 0)
    def _():
        acc[...] = jnp.zeros_like(acc)

    dinv = _dinv_block(p0_ref, p1_ref)
    h = dinv * (q0_ref[...] + q1_ref[...] + y_ref[...]) + b_ref[...]
    gids = lax.broadcasted_iota(jnp.int32, (1, G), 1)
    bt = bt_ref[0, 0, :]
    onehot = (bt[:, None] == gids).astype(jnp.float32)
    ones = jnp.ones((RB, 8), jnp.float32)
    zer = jnp.zeros((RB, 128 - H - 8), jnp.float32)
    hx = jnp.concatenate([h, ones, zer], axis=1)
    acc[...] += lax.dot_general(onehot, hx, (((0,), (0,)), ((), ())),
                                preferred_element_type=jnp.float32)

    @pl.when(i == pl.num_programs(0) - 1)
    def _():
        pooled_sum = acc[:, :H]
        counts = jnp.maximum(acc[:, H:H + 1], 1.0)
        pooled = pooled_sum / counts
        logits = jnp.dot(pooled, fcw_ref[...],
                         preferred_element_type=jnp.float32) + fcb_ref[...]
        m = jnp.max(logits, axis=1, keepdims=True)
        lse = jnp.log(jnp.sum(jnp.exp(logits - m), axis=1, keepdims=True)) + m
        o_ref[...] = logits - lse


def _tc_scale_matmul(x_p, p0d, p1d, W):
    return pl.pallas_call(
        _tc_scale_matmul_body,
        grid=(NRB,),
        in_specs=[
            pl.BlockSpec((RB, D), lambda i: (i, 0)),
            pl.BlockSpec((RB, DEGW), lambda i: (i, 0)),
            pl.BlockSpec((RB, DEGW), lambda i: (i, 0)),
            pl.BlockSpec((D, H), lambda i: (0, 0)),
        ],
        out_specs=pl.BlockSpec((RB, H), lambda i: (i, 0)),
        out_shape=jax.ShapeDtypeStruct((NPAD, H), jnp.float32),
    )(x_p, p0d, p1d, W)


def _tc_fuse(q, y, p0d, p1d, b, W):
    return pl.pallas_call(
        _tc_fuse_body,
        grid=(NRB,),
        in_specs=[
            pl.BlockSpec((RB, H), lambda i: (i, 0)),
            pl.BlockSpec((RB, H), lambda i: (i, 0)),
            pl.BlockSpec((RB, H), lambda i: (i, 0)),
            pl.BlockSpec((RB, DEGW), lambda i: (i, 0)),
            pl.BlockSpec((RB, DEGW), lambda i: (i, 0)),
            pl.BlockSpec((1, H), lambda i: (0, 0)),
            pl.BlockSpec((H, H), lambda i: (0, 0)),
        ],
        out_specs=pl.BlockSpec((RB, H), lambda i: (i, 0)),
        out_shape=jax.ShapeDtypeStruct((NPAD, H), jnp.float32),
    )(q[0], q[1], y, p0d, p1d, b, W)


def _tc_final(q, y2, p0d, p1d, bt3, b2, fcw_p, fcb_p):
    return pl.pallas_call(
        _tc_final_body,
        grid=(NRB,),
        in_specs=[
            pl.BlockSpec((RB, H), lambda i: (i, 0)),
            pl.BlockSpec((RB, H), lambda i: (i, 0)),
            pl.BlockSpec((RB, H), lambda i: (i, 0)),
            pl.BlockSpec((RB, DEGW), lambda i: (i, 0)),
            pl.BlockSpec((RB, DEGW), lambda i: (i, 0)),
            pl.BlockSpec((1, 1, RB), lambda i: (i, 0, 0)),
            pl.BlockSpec((1, H), lambda i: (0, 0)),
            pl.BlockSpec((H, 128), lambda i: (0, 0)),
            pl.BlockSpec((1, 128), lambda i: (0, 0)),
        ],
        out_specs=pl.BlockSpec((G, 128), lambda i: (0, 0)),
        out_shape=jax.ShapeDtypeStruct((G, 128), jnp.float32),
        scratch_shapes=[pltpu.VMEM((G, 128), jnp.float32)],
    )(q[0], q[1], y2, p0d, p1d, bt3, b2, fcw_p, fcb_p)


def kernel(x, edge_index, batch, W1, b1, W2, b2, fcW, fcb):
    pad1 = jnp.full((EPAD - E,), N, dtype=jnp.int32)
    dummy = jnp.full((NW, 1, CH), N, dtype=jnp.int32)
    row3 = jnp.concatenate(
        [jnp.concatenate([edge_index[0], pad1]).reshape(NW, NCH, CH), dummy], 1)
    col3 = jnp.concatenate(
        [jnp.concatenate([edge_index[1], pad1]).reshape(NW, NCH, CH), dummy], 1)

    x_p = jnp.zeros((NPAD, D), jnp.float32).at[:N].set(x)
    bt3 = jnp.full((NPAD,), -1, jnp.int32).at[:N].set(batch).reshape(NRB, 1, RB)

    ones_deg = jnp.ones((CH, DEGW), jnp.float32)
    zeros_deg = jnp.zeros((RPT, DEGW), jnp.float32)
    zeros_agg = jnp.zeros((RPT, H), jnp.float32)

    b1r = b1.reshape(1, H)
    b2r = b2.reshape(1, H)
    fcw_p = jnp.zeros((H, 128), jnp.float32).at[:, :O].set(fcW)
    fcb_p = jnp.full((1, 128), -1e30, jnp.float32).at[0, :O].set(fcb)

    degp = _sc_degree(col3, ones_deg, zeros_deg)
    p0d, p1d = degp[0], degp[1]

    y1 = _tc_scale_matmul(x_p, p0d, p1d, W1)
    q1 = _sc_aggregate(y1, row3, col3, zeros_agg)
    y2 = _tc_fuse(q1, y1, p0d, p1d, b1r, W2)
    q2 = _sc_aggregate(y2, row3, col3, zeros_agg)
    outp = _tc_final(q2, y2, p0d, p1d, bt3, b2r, fcw_p, fcb_p)
    return outp[:, :O]


# speedup vs baseline: 2.8199x; 2.8199x over previous
import functools

import jax
import jax.numpy as jnp
from jax import lax
from jax.experimental import pallas as pl
from jax.experimental.pallas import tpu as pltpu
from jax.experimental.pallas import tpu_sc as plsc

N = 10000
E = 320000
D = 128
H = 64
O = 3
G = 64

NC = 2
NS = 16
NW = NC * NS

CH = 256
NCH = 40
NCH1 = NCH + 1
EPW = NCH * CH
EPAD = NW * EPW

NPAD = 10240
RPT = NPAD // NS
RB = 1024
NRB = NPAD // RB
DEGW = 16

_mesh = plsc.VectorSubcoreMesh(core_axis_name="c", subcore_axis_name="s",
                               num_cores=NC, num_subcores=NS)
_sc_params = pltpu.CompilerParams(use_tc_tiling_on_sc=False)


@functools.partial(
    pl.kernel,
    out_type=jax.ShapeDtypeStruct((NC, NPAD, DEGW), jnp.float32),
    mesh=_mesh,
    compiler_params=_sc_params,
    scratch_types=[
        pltpu.VMEM((NCH1, CH), jnp.int32),
        pltpu.VMEM((CH, DEGW), jnp.float32),
        pltpu.VMEM_SHARED((NPAD, DEGW), jnp.float32),
    ],
)
def _sc_degree(col_hbm, ones_hbm, zeros_hbm, out_hbm, col_v, ones_v, acc_sh):
    c = lax.axis_index("c")
    s = lax.axis_index("s")
    wid = s * NC + c
    pltpu.sync_copy(zeros_hbm, acc_sh.at[pl.ds(s * RPT, RPT)])
    pltpu.sync_copy(ones_hbm, ones_v)
    pltpu.sync_copy(col_hbm.at[wid], col_v)
    plsc.subcore_barrier()

    def body(j, carry):
        pltpu.sync_copy(ones_v, acc_sh.at[col_v.at[j]], add=True)
        return carry

    lax.fori_loop(0, NCH, body, 0)
    plsc.subcore_barrier()
    pltpu.sync_copy(acc_sh.at[pl.ds(s * RPT, RPT)],
                    out_hbm.at[c, pl.ds(s * RPT, RPT)])


@functools.partial(
    pl.kernel,
    out_type=jax.ShapeDtypeStruct((NC, NPAD, H), jnp.float32),
    mesh=_mesh,
    compiler_params=_sc_params,
    scratch_types=[
        pltpu.VMEM((NCH1, CH), jnp.int32),
        pltpu.VMEM((NCH1, CH), jnp.int32),
        pltpu.VMEM((CH, H), jnp.float32),
        pltpu.VMEM_SHARED((NPAD, H), jnp.float32),
        pltpu.VMEM_SHARED((NPAD, H), jnp.float32),
        pltpu.SemaphoreType.DMA,
    ],
)
def _sc_aggregate(y_hbm, row_hbm, col_hbm, zeros_hbm, out_hbm,
                  row_v, col_v, buf, y_sh, acc_sh, sem):
    c = lax.axis_index("c")
    s = lax.axis_index("s")
    wid = s * NC + c
    pltpu.sync_copy(zeros_hbm, acc_sh.at[pl.ds(s * RPT, RPT)])
    pltpu.sync_copy(y_hbm.at[pl.ds(s * RPT, RPT)], y_sh.at[pl.ds(s * RPT, RPT)])
    pltpu.sync_copy(row_hbm.at[wid], row_v)
    pltpu.sync_copy(col_hbm.at[wid], col_v)
    plsc.subcore_barrier()

    def body(j, carry):
        pltpu.async_copy(y_sh.at[row_v.at[j]], buf, sem).wait()
        pltpu.sync_copy(buf, acc_sh.at[col_v.at[j]], add=True)
        return carry

    lax.fori_loop(0, NCH, body, 0)
    plsc.subcore_barrier()
    pltpu.sync_copy(acc_sh.at[pl.ds(s * RPT, RPT)],
                    out_hbm.at[c, pl.ds(s * RPT, RPT)])


def _dinv_block(p0, p1):
    deg = 1.0 + p0[:, 0:1] + p1[:, 0:1]
    return lax.rsqrt(deg)


def _tc_scale_matmul_body(x_ref, p0_ref, p1_ref, w_ref, o_ref):
    dinv = _dinv_block(p0_ref, p1_ref)
    xl = jnp.dot(x_ref[...], w_ref[...], preferred_element_type=jnp.float32)
    o_ref[...] = xl * dinv


def _tc_fuse_body(q0_ref, q1_ref, y_ref, p0_ref, p1_ref, b_ref, w_ref, o_ref):
    dinv = _dinv_block(p0_ref, p1_ref)
    h = dinv * (q0_ref[...] + q1_ref[...] + y_ref[...]) + b_ref[...]
    h = jnp.maximum(h, 0.0)
    o_ref[...] = jnp.dot(h, w_ref[...], preferred_element_type=jnp.float32) * dinv


def _tc_final_body(q0_ref, q1_ref, y_ref, p0_ref, p1_ref, bt_ref, b_ref,
                   fcw_ref, fcb_ref, o_ref, acc):
    i = pl.program_id(0)

    @pl.when(i == 0)
    def _():
        acc[...] = jnp.zeros_like(acc)

    dinv = _dinv_block(p0_ref, p1_ref)
    h = dinv * (q0_ref[...] + q1_ref[...] + y_ref[...]) + b_ref[...]
    gids = lax.broadcasted_iota(jnp.int32, (1, G), 1)
    bt = bt_ref[0, 0, :]
    onehot = (bt[:, None] == gids).astype(jnp.float32)
    ones = jnp.ones((RB, 8), jnp.float32)
    zer = jnp.zeros((RB, 128 - H - 8), jnp.float32)
    hx = jnp.concatenate([h, ones, zer], axis=1)
    acc[...] += lax.dot_general(onehot, hx, (((0,), (0,)), ((), ())),
                                preferred_element_type=jnp.float32)

    @pl.when(i == pl.num_programs(0) - 1)
    def _():
        pooled_sum = acc[:, :H]
        counts = jnp.maximum(acc[:, H:H + 1], 1.0)
        pooled = pooled_sum / counts
        logits = jnp.dot(pooled, fcw_ref[...],
                         preferred_element_type=jnp.float32) + fcb_ref[...]
        m = jnp.max(logits, axis=1, keepdims=True)
        lse = jnp.log(jnp.sum(jnp.exp(logits - m), axis=1, keepdims=True)) + m
        o_ref[...] = logits - lse


def _tc_scale_matmul(x_p, p0d, p1d, W):
    return pl.pallas_call(
        _tc_scale_matmul_body,
        grid=(NRB,),
        in_specs=[
            pl.BlockSpec((RB, D), lambda i: (i, 0)),
            pl.BlockSpec((RB, DEGW), lambda i: (i, 0)),
            pl.BlockSpec((RB, DEGW), lambda i: (i, 0)),
            pl.BlockSpec((D, H), lambda i: (0, 0)),
        ],
        out_specs=pl.BlockSpec((RB, H), lambda i: (i, 0)),
        out_shape=jax.ShapeDtypeStruct((NPAD, H), jnp.float32),
    )(x_p, p0d, p1d, W)


def _tc_fuse(q, y, p0d, p1d, b, W):
    return pl.pallas_call(
        _tc_fuse_body,
        grid=(NRB,),
        in_specs=[
            pl.BlockSpec((RB, H), lambda i: (i, 0)),
            pl.BlockSpec((RB, H), lambda i: (i, 0)),
            pl.BlockSpec((RB, H), lambda i: (i, 0)),
            pl.BlockSpec((RB, DEGW), lambda i: (i, 0)),
            pl.BlockSpec((RB, DEGW), lambda i: (i, 0)),
            pl.BlockSpec((1, H), lambda i: (0, 0)),
            pl.BlockSpec((H, H), lambda i: (0, 0)),
        ],
        out_specs=pl.BlockSpec((RB, H), lambda i: (i, 0)),
        out_shape=jax.ShapeDtypeStruct((NPAD, H), jnp.float32),
    )(q[0], q[1], y, p0d, p1d, b, W)


def _tc_final(q, y2, p0d, p1d, bt3, b2, fcw_p, fcb_p):
    return pl.pallas_call(
        _tc_final_body,
        grid=(NRB,),
        in_specs=[
            pl.BlockSpec((RB, H), lambda i: (i, 0)),
            pl.BlockSpec((RB, H), lambda i: (i, 0)),
            pl.BlockSpec((RB, H), lambda i: (i, 0)),
            pl.BlockSpec((RB, DEGW), lambda i: (i, 0)),
            pl.BlockSpec((RB, DEGW), lambda i: (i, 0)),
            pl.BlockSpec((1, 1, RB), lambda i: (i, 0, 0)),
            pl.BlockSpec((1, H), lambda i: (0, 0)),
            pl.BlockSpec((H, 128), lambda i: (0, 0)),
            pl.BlockSpec((1, 128), lambda i: (0, 0)),
        ],
        out_specs=pl.BlockSpec((G, 128), lambda i: (0, 0)),
        out_shape=jax.ShapeDtypeStruct((G, 128), jnp.float32),
        scratch_shapes=[pltpu.VMEM((G, 128), jnp.float32)],
    )(q[0], q[1], y2, p0d, p1d, bt3, b2, fcw_p, fcb_p)


def kernel(x, edge_index, batch, W1, b1, W2, b2, fcW, fcb):
    pad1 = jnp.full((EPAD - E,), N, dtype=jnp.int32)
    dummy = jnp.full((NW, 1, CH), N, dtype=jnp.int32)
    row3 = jnp.concatenate(
        [jnp.concatenate([edge_index[0], pad1]).reshape(NW, NCH, CH), dummy], 1)
    col3 = jnp.concatenate(
        [jnp.concatenate([edge_index[1], pad1]).reshape(NW, NCH, CH), dummy], 1)

    x_p = jnp.zeros((NPAD, D), jnp.float32).at[:N].set(x)
    bt3 = jnp.full((NPAD,), -1, jnp.int32).at[:N].set(batch).reshape(NRB, 1, RB)

    ones_deg = jnp.ones((CH, DEGW), jnp.float32)
    zeros_deg = jnp.zeros((RPT, DEGW), jnp.float32)
    zeros_agg = jnp.zeros((RPT, H), jnp.float32)

    b1r = b1.reshape(1, H)
    b2r = b2.reshape(1, H)
    fcw_p = jnp.zeros((H, 128), jnp.float32).at[:, :O].set(fcW)
    fcb_p = jnp.full((1, 128), -1e30, jnp.float32).at[0, :O].set(fcb)

    degp = _sc_degree(col3, ones_deg, zeros_deg)
    p0d, p1d = degp[0], degp[1]

    y1 = _tc_scale_matmul(x_p, p0d, p1d, W1)
    q1 = _sc_aggregate(y1, row3, col3, zeros_agg)
    y2 = _tc_fuse(q1, y1, p0d, p1d, b1r, W2)
    q2 = _sc_aggregate(y2, row3, col3, zeros_agg)
    outp = _tc_final(q2, y2, p0d, p1d, bt3, b2r, fcw_p, fcb_p)
    return outp[:, :O]


# speedup vs baseline: 2.9929x; 1.0613x over previous
import functools

import jax
import jax.numpy as jnp
from jax import lax
from jax.experimental import pallas as pl
from jax.experimental.pallas import tpu as pltpu
from jax.experimental.pallas import tpu_sc as plsc

N = 10000
E = 320000
D = 128
H = 64
O = 3
G = 64

NC = 2
NS = 16
NW = NC * NS

CH = 256
NCH = 40
NCH1 = NCH + 1
EPW = NCH * CH
EPAD = NW * EPW

NPAD = 10240
RPT = NPAD // NS
RB = 1024
NRB = NPAD // RB
DEGW = 16

_mesh = plsc.VectorSubcoreMesh(core_axis_name="c", subcore_axis_name="s",
                               num_cores=NC, num_subcores=NS)
_sc_params = pltpu.CompilerParams(use_tc_tiling_on_sc=False)


@functools.partial(
    pl.kernel,
    out_type=jax.ShapeDtypeStruct((NC, NPAD, DEGW), jnp.float32),
    mesh=_mesh,
    compiler_params=_sc_params,
    scratch_types=[
        pltpu.VMEM((NCH1, CH), jnp.int32),
        pltpu.VMEM((CH, DEGW), jnp.float32),
        pltpu.VMEM_SHARED((NPAD, DEGW), jnp.float32),
    ],
)
def _sc_degree(col_hbm, ones_hbm, zeros_hbm, out_hbm, col_v, ones_v, acc_sh):
    c = lax.axis_index("c")
    s = lax.axis_index("s")
    wid = s * NC + c
    pltpu.sync_copy(zeros_hbm, acc_sh.at[pl.ds(s * RPT, RPT)])
    pltpu.sync_copy(ones_hbm, ones_v)
    pltpu.sync_copy(col_hbm.at[wid], col_v)
    plsc.subcore_barrier()

    def body(j, carry):
        pltpu.sync_copy(ones_v, acc_sh.at[col_v.at[j]], add=True)
        return carry

    lax.fori_loop(0, NCH, body, 0)
    plsc.subcore_barrier()
    pltpu.sync_copy(acc_sh.at[pl.ds(s * RPT, RPT)],
                    out_hbm.at[c, pl.ds(s * RPT, RPT)])


@functools.partial(
    pl.kernel,
    out_type=jax.ShapeDtypeStruct((NC, NPAD, H), jnp.float32),
    mesh=_mesh,
    compiler_params=_sc_params,
    scratch_types=[
        pltpu.VMEM((NCH1, CH), jnp.int32),
        pltpu.VMEM((NCH1, CH), jnp.int32),
        pltpu.VMEM((CH, H), jnp.float32),
        pltpu.VMEM_SHARED((NPAD, H), jnp.float32),
        pltpu.VMEM_SHARED((NPAD, H), jnp.float32),
        pltpu.SemaphoreType.DMA,
    ],
)
def _sc_aggregate(y_hbm, row_hbm, col_hbm, zeros_hbm, out_hbm,
                  row_v, col_v, buf, y_sh, acc_sh, sem):
    c = lax.axis_index("c")
    s = lax.axis_index("s")
    wid = s * NC + c
    pltpu.sync_copy(zeros_hbm, acc_sh.at[pl.ds(s * RPT, RPT)])
    pltpu.sync_copy(y_hbm.at[pl.ds(s * RPT, RPT)], y_sh.at[pl.ds(s * RPT, RPT)])
    pltpu.sync_copy(row_hbm.at[wid], row_v)
    pltpu.sync_copy(col_hbm.at[wid], col_v)
    plsc.subcore_barrier()

    def body(j, carry):
        pltpu.async_copy(y_sh.at[row_v.at[j]], buf, sem).wait()
        pltpu.sync_copy(buf, acc_sh.at[col_v.at[j]], add=True)
        return carry

    lax.fori_loop(0, NCH, body, 0)
    plsc.subcore_barrier()
    pltpu.sync_copy(acc_sh.at[pl.ds(s * RPT, RPT)],
                    out_hbm.at[c, pl.ds(s * RPT, RPT)])


def _dinv_block(p_ref):
    deg = 1.0 + p_ref[0, :, 0:1] + p_ref[1, :, 0:1]
    return lax.rsqrt(deg)


def _tc_scale_matmul_body(x_ref, p_ref, w_ref, o_ref):
    dinv = _dinv_block(p_ref)
    xl = jnp.dot(x_ref[...], w_ref[...], preferred_element_type=jnp.float32)
    o_ref[...] = xl * dinv


def _tc_fuse_body(q_ref, y_ref, p_ref, b_ref, w_ref, o_ref):
    dinv = _dinv_block(p_ref)
    h = dinv * (q_ref[0] + q_ref[1] + y_ref[...]) + b_ref[...]
    h = jnp.maximum(h, 0.0)
    o_ref[...] = jnp.dot(h, w_ref[...], preferred_element_type=jnp.float32) * dinv


def _tc_final_body(q_ref, y_ref, p_ref, bt_ref, b_ref,
                   fcw_ref, fcb_ref, o_ref, acc):
    i = pl.program_id(0)

    @pl.when(i == 0)
    def _():
        acc[...] = jnp.zeros_like(acc)

    dinv = _dinv_block(p_ref)
    h = dinv * (q_ref[0] + q_ref[1] + y_ref[...]) + b_ref[...]
    gids = lax.broadcasted_iota(jnp.int32, (1, G), 1)
    bt = bt_ref[0, 0, :]
    onehot = (bt[:, None] == gids).astype(jnp.float32)
    ones = jnp.ones((RB, 8), jnp.float32)
    zer = jnp.zeros((RB, 128 - H - 8), jnp.float32)
    hx = jnp.concatenate([h, ones, zer], axis=1)
    acc[...] += lax.dot_general(onehot, hx, (((0,), (0,)), ((), ())),
                                preferred_element_type=jnp.float32)

    @pl.when(i == pl.num_programs(0) - 1)
    def _():
        pooled_sum = acc[:, :H]
        counts = jnp.maximum(acc[:, H:H + 1], 1.0)
        pooled = pooled_sum / counts
        logits = jnp.dot(pooled, fcw_ref[...],
                         preferred_element_type=jnp.float32) + fcb_ref[...]
        m = jnp.max(logits, axis=1, keepdims=True)
        lse = jnp.log(jnp.sum(jnp.exp(logits - m), axis=1, keepdims=True)) + m
        o_ref[...] = logits - lse


def _tc_scale_matmul(x_p, degp, W):
    return pl.pallas_call(
        _tc_scale_matmul_body,
        grid=(NRB,),
        in_specs=[
            pl.BlockSpec((RB, D), lambda i: (i, 0)),
            pl.BlockSpec((2, RB, DEGW), lambda i: (0, i, 0)),
            pl.BlockSpec((D, H), lambda i: (0, 0)),
        ],
        out_specs=pl.BlockSpec((RB, H), lambda i: (i, 0)),
        out_shape=jax.ShapeDtypeStruct((NPAD, H), jnp.float32),
    )(x_p, degp, W)


def _tc_fuse(q, y, degp, b, W):
    return pl.pallas_call(
        _tc_fuse_body,
        grid=(NRB,),
        in_specs=[
            pl.BlockSpec((2, RB, H), lambda i: (0, i, 0)),
            pl.BlockSpec((RB, H), lambda i: (i, 0)),
            pl.BlockSpec((2, RB, DEGW), lambda i: (0, i, 0)),
            pl.BlockSpec((1, H), lambda i: (0, 0)),
            pl.BlockSpec((H, H), lambda i: (0, 0)),
        ],
        out_specs=pl.BlockSpec((RB, H), lambda i: (i, 0)),
        out_shape=jax.ShapeDtypeStruct((NPAD, H), jnp.float32),
    )(q, y, degp, b, W)


def _tc_final(q, y2, degp, bt3, b2, fcw_p, fcb_p):
    return pl.pallas_call(
        _tc_final_body,
        grid=(NRB,),
        in_specs=[
            pl.BlockSpec((2, RB, H), lambda i: (0, i, 0)),
            pl.BlockSpec((RB, H), lambda i: (i, 0)),
            pl.BlockSpec((2, RB, DEGW), lambda i: (0, i, 0)),
            pl.BlockSpec((1, 1, RB), lambda i: (i, 0, 0)),
            pl.BlockSpec((1, H), lambda i: (0, 0)),
            pl.BlockSpec((H, 128), lambda i: (0, 0)),
            pl.BlockSpec((1, 128), lambda i: (0, 0)),
        ],
        out_specs=pl.BlockSpec((G, 128), lambda i: (0, 0)),
        out_shape=jax.ShapeDtypeStruct((G, 128), jnp.float32),
        scratch_shapes=[pltpu.VMEM((G, 128), jnp.float32)],
    )(q, y2, degp, bt3, b2, fcw_p, fcb_p)


def kernel(x, edge_index, batch, W1, b1, W2, b2, fcW, fcb):
    pad1 = jnp.full((EPAD - E,), N, dtype=jnp.int32)
    dummy = jnp.full((NW, 1, CH), N, dtype=jnp.int32)
    row3 = jnp.concatenate(
        [jnp.concatenate([edge_index[0], pad1]).reshape(NW, NCH, CH), dummy], 1)
    col3 = jnp.concatenate(
        [jnp.concatenate([edge_index[1], pad1]).reshape(NW, NCH, CH), dummy], 1)

    x_p = jnp.zeros((NPAD, D), jnp.float32).at[:N].set(x)
    bt3 = jnp.full((NPAD,), -1, jnp.int32).at[:N].set(batch).reshape(NRB, 1, RB)

    ones_deg = jnp.ones((CH, DEGW), jnp.float32)
    zeros_deg = jnp.zeros((RPT, DEGW), jnp.float32)
    zeros_agg = jnp.zeros((RPT, H), jnp.float32)

    b1r = b1.reshape(1, H)
    b2r = b2.reshape(1, H)
    fcw_p = jnp.zeros((H, 128), jnp.float32).at[:, :O].set(fcW)
    fcb_p = jnp.full((1, 128), -1e30, jnp.float32).at[0, :O].set(fcb)

    degp = _sc_degree(col3, ones_deg, zeros_deg)

    y1 = _tc_scale_matmul(x_p, degp, W1)
    q1 = _sc_aggregate(y1, row3, col3, zeros_agg)
    y2 = _tc_fuse(q1, y1, degp, b1r, W2)
    q2 = _sc_aggregate(y2, row3, col3, zeros_agg)
    outp = _tc_final(q2, y2, degp, bt3, b2r, fcw_p, fcb_p)
    return outp[:, :O]
